# baseline (speedup 1.0000x reference)
import functools

import jax
import jax.numpy as jnp
from jax import lax
from jax.experimental import pallas as pl
from jax.experimental.pallas import tpu as pltpu
from jax.experimental.pallas import tpu_sc as plsc

_N_EMB = 8192
_DIM = 32
_TM = 1024
_RC = 16
_LC = 128


def _argmin_body(z_ref, emb_ref, idx_ref, esq_ref, ez_ref):
    @pl.when(pl.program_id(0) == 0)
    def _():
        emb = emb_ref[...]
        esq_ref[...] = jnp.broadcast_to(
            jnp.sum(emb * emb, axis=1)[None, :], (8, _N_EMB))

    z = z_ref[...]
    ez_ref[...] = lax.dot_general(
        z * (-2.0), emb_ref[...], (((1,), (1,)), ((), ())),
        preferred_element_type=jnp.float32).reshape(
            _TM // 8, 8, _N_EMB)

    ncols = _N_EMB // _LC
    nchunks = _TM // _RC
    rb = _RC // 8
    big = jnp.float32(_N_EMB)

    def chunk_body(i, carry):
        zc = z_ref[pl.ds(i * _RC, _RC), :].reshape(rb, 8, _DIM)
        z_sq = jnp.sum(zc * zc, axis=2, keepdims=True)
        zsb = jnp.broadcast_to(z_sq, (rb, 8, _LC))
        m = jnp.full((rb, 8, _LC), jnp.inf, jnp.float32)
        jc = jnp.zeros((rb, 8, _LC), jnp.float32)
        for j in range(ncols):
            esq_j = esq_ref[:, pl.ds(j * _LC, _LC)][None]
            ez_j = ez_ref[pl.ds(i * rb, rb), :, pl.ds(j * _LC, _LC)]
            d_j = (zsb + esq_j) + ez_j
            cmp = d_j < m
            m = jnp.where(cmp, d_j, m)
            jc = jnp.where(cmp, jnp.float32(j), jc)
        xm = jnp.min(m, axis=2, keepdims=True)
        lane = lax.broadcasted_iota(
            jnp.int32, (rb, 8, _LC), 2).astype(jnp.float32)
        full = jc * jnp.float32(_LC) + lane
        cand = jnp.where(m == xm, full, big)
        idx_c = jnp.min(cand, axis=2)
        idx_ref[pl.ds(i * rb, rb), :] = idx_c.astype(jnp.int32)
        return carry

    lax.fori_loop(0, nchunks, chunk_body, 0, unroll=2)


def _argmin_call(z_flat, emb):
    m = z_flat.shape[0]
    grid = m // _TM
    return pl.pallas_call(
        _argmin_body,
        grid=(grid,),
        in_specs=[
            pl.BlockSpec((_TM, _DIM), lambda i: (i, 0)),
            pl.BlockSpec((_N_EMB, _DIM), lambda i: (0, 0)),
        ],
        out_specs=pl.BlockSpec((_TM // 8, 8), lambda i: (i, 0)),
        out_shape=jax.ShapeDtypeStruct((m // 8, 8), jnp.int32),
        scratch_shapes=[pltpu.VMEM((8, _N_EMB), jnp.float32),
                        pltpu.VMEM((_TM // 8, 8, _N_EMB), jnp.float32)],
    )(z_flat, emb)


_NC = 2
_NW = 32


@functools.lru_cache(maxsize=None)
def _make_gather(b_total):
    b_per_w = b_total // _NW
    mesh = plsc.VectorSubcoreMesh(core_axis_name="c", subcore_axis_name="s")

    @functools.partial(
        pl.kernel, mesh=mesh,
        out_type=jax.ShapeDtypeStruct((b_total, _DIM), jnp.float32),
        scratch_types=[
            pltpu.VMEM((b_per_w,), jnp.int32),
            pltpu.VMEM((b_per_w, _DIM), jnp.float32),
            pltpu.SemaphoreType.DMA,
        ],
        compiler_params=pltpu.CompilerParams(use_tc_tiling_on_sc=False),
    )
    def gather(table_hbm, idx_hbm, out_hbm, idx_v, rows_v, sem):
        wid = lax.axis_index("s") * _NC + lax.axis_index("c")
        base = wid * b_per_w
        pltpu.sync_copy(idx_hbm.at[pl.ds(base, b_per_w)], idx_v)
        pltpu.async_copy(table_hbm.at[idx_v], rows_v, sem).wait()
        pltpu.sync_copy(rows_v, out_hbm.at[pl.ds(base, b_per_w)])

    return gather


def kernel(z, embedding_weight):
    bs, c, n = z.shape
    z_flat = jnp.transpose(z, (0, 2, 1)).reshape(bs * n, c)
    idx = _argmin_call(z_flat, embedding_weight).reshape(bs * n)
    z_q_flat = _make_gather(bs * n)(embedding_weight, idx)
    return jnp.transpose(z_q_flat.reshape(bs, n, c), (0, 2, 1))


# speedup vs baseline: 1.7253x; 1.7253x over previous
import functools

import jax
import jax.numpy as jnp
from jax import lax
from jax.experimental import pallas as pl
from jax.experimental.pallas import tpu as pltpu
from jax.experimental.pallas import tpu_sc as plsc

_N_EMB = 8192
_DIM = 32
_TM = 1024


def _argmin_body(z_ref, emb_ref, idx_ref, esq_ref, ids_ref, d_ref):
    @pl.when(pl.program_id(0) == 0)
    def _():
        emb = emb_ref[...]
        esq_ref[...] = jnp.broadcast_to(
            jnp.sum(emb * emb, axis=1)[None, :], (8, _N_EMB))
        ids_ref[...] = lax.broadcasted_iota(
            jnp.int32, (8, _N_EMB), 1).astype(jnp.float32)

    z = z_ref[...]
    z3 = z.reshape(_TM // 8, 8, _DIM)
    z_sq = jnp.sum(z3 * z3, axis=2, keepdims=True)
    neg2_e_z = lax.dot_general(
        z * (-2.0), emb_ref[...], (((1,), (1,)), ((), ())),
        preferred_element_type=jnp.float32)
    d = (z_sq + esq_ref[...][None]) + neg2_e_z.reshape(_TM // 8, 8, _N_EMB)
    d_ref[...] = d
    min_v = jnp.min(d, axis=2, keepdims=True)
    idx = jnp.min(
        jnp.where(d_ref[...] == min_v, ids_ref[...][None],
                  jnp.float32(_N_EMB)),
        axis=2)
    idx_ref[...] = idx.astype(jnp.int32)


def _argmin_call(z_flat, emb):
    m = z_flat.shape[0]
    grid = m // _TM
    return pl.pallas_call(
        _argmin_body,
        grid=(grid,),
        in_specs=[
            pl.BlockSpec((_TM, _DIM), lambda i: (i, 0)),
            pl.BlockSpec((_N_EMB, _DIM), lambda i: (0, 0)),
        ],
        out_specs=pl.BlockSpec((_TM // 8, 8), lambda i: (i, 0)),
        out_shape=jax.ShapeDtypeStruct((m // 8, 8), jnp.int32),
        scratch_shapes=[pltpu.VMEM((8, _N_EMB), jnp.float32),
                        pltpu.VMEM((8, _N_EMB), jnp.float32),
                        pltpu.VMEM((_TM // 8, 8, _N_EMB), jnp.float32)],
    )(z_flat, emb)


_NC = 2
_NW = 32


@functools.lru_cache(maxsize=None)
def _make_gather(b_total):
    b_per_w = b_total // _NW
    mesh = plsc.VectorSubcoreMesh(core_axis_name="c", subcore_axis_name="s")

    @functools.partial(
        pl.kernel, mesh=mesh,
        out_type=jax.ShapeDtypeStruct((b_total, _DIM), jnp.float32),
        scratch_types=[
            pltpu.VMEM((b_per_w,), jnp.int32),
            pltpu.VMEM((b_per_w, _DIM), jnp.float32),
            pltpu.SemaphoreType.DMA,
        ],
        compiler_params=pltpu.CompilerParams(use_tc_tiling_on_sc=False),
    )
    def gather(table_hbm, idx_hbm, out_hbm, idx_v, rows_v, sem):
        wid = lax.axis_index("s") * _NC + lax.axis_index("c")
        base = wid * b_per_w
        pltpu.sync_copy(idx_hbm.at[pl.ds(base, b_per_w)], idx_v)
        pltpu.async_copy(table_hbm.at[idx_v], rows_v, sem).wait()
        pltpu.sync_copy(rows_v, out_hbm.at[pl.ds(base, b_per_w)])

    return gather


def kernel(z, embedding_weight):
    bs, c, n = z.shape
    z_flat = jnp.transpose(z, (0, 2, 1)).reshape(bs * n, c)
    idx = _argmin_call(z_flat, embedding_weight).reshape(bs * n)
    z_q_flat = _make_gather(bs * n)(embedding_weight, idx)
    return jnp.transpose(z_q_flat.reshape(bs, n, c), (0, 2, 1))
